# baseline (device time: 86252 ns/iter reference)
import jax
import jax.numpy as jnp
from jax import lax
from jax.experimental import pallas as pl
from jax.experimental.pallas import tpu as pltpu

N_DEV = 4
E_LOCAL = 4


def kernel(x, router_W, route_idx, expert_W, shared_W):
    n_tok, d_model = x.shape
    n_exp = router_W.shape[1]
    d_ff = expert_W.shape[2]

    def body(x_ref, rw_ref, ridx_ref, ew_ref, sw_ref, out_ref,
             comm_ref, send_sems, recv_sems):
        my_pos = lax.axis_index("i")
        left = (my_pos - 1) % N_DEV
        right = (my_pos + 1) % N_DEV

        barrier_sem = pltpu.get_barrier_semaphore()
        for nbr in [left, right]:
            pl.semaphore_signal(
                barrier_sem, inc=1,
                device_id=(nbr,), device_id_type=pl.DeviceIdType.MESH,
            )
        pl.semaphore_wait(barrier_sem, 2)

        xv = x_ref[:, :]
        scores = jnp.dot(xv, rw_ref[:, :], preferred_element_type=jnp.float32)
        s_max = jnp.max(scores, axis=1, keepdims=True)
        p = jnp.exp(scores - s_max)
        probs = p / jnp.sum(p, axis=1, keepdims=True)
        idx = ridx_ref[:, :]
        eids = lax.broadcasted_iota(jnp.int32, (n_tok, n_exp), 1)
        w = jnp.sum(jnp.where(eids == idx, probs, 0.0), axis=1, keepdims=True)

        partial = jnp.zeros((n_tok, d_ff), jnp.float32)
        for le in range(E_LOCAL):
            gate = jnp.where(idx == my_pos * E_LOCAL + le, w, 0.0)
            partial = partial + jnp.dot(
                xv * gate, ew_ref[le], preferred_element_type=jnp.float32
            )
        comm_ref[0, :, :] = partial

        shared = jnp.dot(xv, sw_ref[:, :], preferred_element_type=jnp.float32)
        out_ref[:, :] = shared + partial

        for h in range(N_DEV - 1):
            rdma = pltpu.make_async_remote_copy(
                src_ref=comm_ref.at[h],
                dst_ref=comm_ref.at[h + 1],
                send_sem=send_sems.at[h],
                recv_sem=recv_sems.at[h],
                device_id=(right,),
                device_id_type=pl.DeviceIdType.MESH,
            )
            rdma.start()
            rdma.wait()
            out_ref[:, :] += comm_ref[h + 1, :, :]

    return pl.pallas_call(
        body,
        out_shape=jax.ShapeDtypeStruct((n_tok, d_ff), jnp.float32),
        in_specs=[pl.BlockSpec(memory_space=pltpu.VMEM)] * 5,
        out_specs=pl.BlockSpec(memory_space=pltpu.VMEM),
        scratch_shapes=[
            pltpu.VMEM((N_DEV, n_tok, d_ff), jnp.float32),
            pltpu.SemaphoreType.DMA((N_DEV - 1,)),
            pltpu.SemaphoreType.DMA((N_DEV - 1,)),
        ],
        compiler_params=pltpu.CompilerParams(collective_id=0),
    )(x, router_W, route_idx, expert_W, shared_W)


# device time: 39781 ns/iter; 2.1682x vs baseline; 2.1682x over previous
import jax
import jax.numpy as jnp
from jax import lax
from jax.experimental import pallas as pl
from jax.experimental.pallas import tpu as pltpu

N_DEV = 4
E_LOCAL = 4


def kernel(x, router_W, route_idx, expert_W, shared_W):
    n_tok, d_model = x.shape
    n_exp = router_W.shape[1]
    d_ff = expert_W.shape[2]
    d_half = d_ff // 2

    def body(x_ref, rw_ref, ridx_ref, ew_ref, sw_ref, out_ref,
             commA, commB, sendA, recvA, sendB, recvB):
        my_pos = lax.axis_index("i")
        px = 3 - my_pos
        py = my_pos ^ 1

        barrier_sem = pltpu.get_barrier_semaphore()
        for nbr in [px, py]:
            pl.semaphore_signal(
                barrier_sem, inc=1,
                device_id=(nbr,), device_id_type=pl.DeviceIdType.MESH,
            )
        pl.semaphore_wait(barrier_sem, 2)

        xv = x_ref[:, :]
        scores = jnp.dot(xv, rw_ref[:, :], preferred_element_type=jnp.float32)
        s_max = jnp.max(scores, axis=1, keepdims=True)
        p = jnp.exp(scores - s_max)
        probs = p / jnp.sum(p, axis=1, keepdims=True)
        idx = ridx_ref[:, :]
        eids = lax.broadcasted_iota(jnp.int32, (n_tok, n_exp), 1)
        w = jnp.sum(jnp.where(eids == idx, probs, 0.0), axis=1, keepdims=True)

        xg = [
            xv * jnp.where(idx == my_pos * E_LOCAL + le, w, 0.0)
            for le in range(E_LOCAL)
        ]

        def exchange(comm, slot_src, slot_dst, send_sems, recv_sems, tgt, s):
            return pltpu.make_async_remote_copy(
                src_ref=comm.at[slot_src],
                dst_ref=comm.at[slot_dst],
                send_sem=send_sems.at[s],
                recv_sem=recv_sems.at[s],
                device_id=(tgt,),
                device_id_type=pl.DeviceIdType.MESH,
            )

        pA = jnp.zeros((n_tok, d_half), jnp.float32)
        for le in range(E_LOCAL):
            pA = pA + jnp.dot(xg[le], ew_ref[le, :, 0:d_half],
                              preferred_element_type=jnp.float32)
        commA[0, :, :] = pA
        rA1 = exchange(commA, 0, 1, sendA, recvA, px, 0)
        rA1.start()

        pB = jnp.zeros((n_tok, d_half), jnp.float32)
        for le in range(E_LOCAL):
            pB = pB + jnp.dot(xg[le], ew_ref[le, :, d_half:d_ff],
                              preferred_element_type=jnp.float32)
        commB[0, :, :] = pB
        rB1 = exchange(commB, 0, 1, sendB, recvB, py, 0)
        rB1.start()

        shared = jnp.dot(xv, sw_ref[:, :], preferred_element_type=jnp.float32)

        rA1.wait_recv()
        commA[2, :, :] = commA[0, :, :] + commA[1, :, :]
        rA2 = exchange(commA, 2, 3, sendA, recvA, py, 1)
        rA2.start()

        rB1.wait_recv()
        commB[2, :, :] = commB[0, :, :] + commB[1, :, :]
        rB2 = exchange(commB, 2, 3, sendB, recvB, px, 1)
        rB2.start()

        rA2.wait_recv()
        out_ref[:, 0:d_half] = (
            shared[:, 0:d_half] + commA[2, :, :] + commA[3, :, :]
        )
        rB2.wait_recv()
        out_ref[:, d_half:d_ff] = (
            shared[:, d_half:d_ff] + commB[2, :, :] + commB[3, :, :]
        )

        rA1.wait_send()
        rB1.wait_send()
        rA2.wait_send()
        rB2.wait_send()

    return pl.pallas_call(
        body,
        out_shape=jax.ShapeDtypeStruct((n_tok, d_ff), jnp.float32),
        in_specs=[pl.BlockSpec(memory_space=pltpu.VMEM)] * 5,
        out_specs=pl.BlockSpec(memory_space=pltpu.VMEM),
        scratch_shapes=[
            pltpu.VMEM((4, n_tok, d_half), jnp.float32),
            pltpu.VMEM((4, n_tok, d_half), jnp.float32),
            pltpu.SemaphoreType.DMA((2,)),
            pltpu.SemaphoreType.DMA((2,)),
            pltpu.SemaphoreType.DMA((2,)),
            pltpu.SemaphoreType.DMA((2,)),
        ],
        compiler_params=pltpu.CompilerParams(collective_id=0),
    )(x, router_W, route_idx, expert_W, shared_W)


# device time: 28307 ns/iter; 3.0470x vs baseline; 1.4053x over previous
import jax
import jax.numpy as jnp
from jax import lax
from jax.experimental import pallas as pl
from jax.experimental.pallas import tpu as pltpu

N_DEV = 4
E_LOCAL = 4


def kernel(x, router_W, route_idx, expert_W, shared_W):
    n_tok, d_model = x.shape
    n_exp = router_W.shape[1]
    d_ff = expert_W.shape[2]
    d_half = d_ff // 2

    def body(x_ref, rw_ref, ridx_ref, ew_ref, sw_ref, out_ref,
             commA, commB, sendA, recvA, sendB, recvB):
        my_pos = lax.axis_index("i")
        px = 3 - my_pos
        py = my_pos ^ 1

        barrier_sem = pltpu.get_barrier_semaphore()
        for nbr in [px, py]:
            pl.semaphore_signal(
                barrier_sem, inc=1,
                device_id=(nbr,), device_id_type=pl.DeviceIdType.MESH,
            )
        pl.semaphore_wait(barrier_sem, 2)

        xv = x_ref[:, :]
        scores = jnp.dot(xv, rw_ref[:, :], preferred_element_type=jnp.float32)
        s_max = jnp.max(scores, axis=1, keepdims=True)
        p = jnp.exp(scores - s_max)
        probs = p / jnp.sum(p, axis=1, keepdims=True)
        idx = ridx_ref[:, :]
        eids = lax.broadcasted_iota(jnp.int32, (n_tok, n_exp), 1)
        w = jnp.sum(jnp.where(eids == idx, probs, 0.0), axis=1, keepdims=True)

        xg = [
            (xv * jnp.where(idx == my_pos * E_LOCAL + le, w, 0.0)).astype(
                jnp.bfloat16
            )
            for le in range(E_LOCAL)
        ]
        xv16 = xv.astype(jnp.bfloat16)

        def exchange(comm, slot_src, slot_dst, send_sems, recv_sems, tgt, s):
            return pltpu.make_async_remote_copy(
                src_ref=comm.at[slot_src],
                dst_ref=comm.at[slot_dst],
                send_sem=send_sems.at[s],
                recv_sem=recv_sems.at[s],
                device_id=(tgt,),
                device_id_type=pl.DeviceIdType.MESH,
            )

        pA = jnp.zeros((n_tok, d_half), jnp.float32)
        for le in range(E_LOCAL):
            pA = pA + jnp.dot(
                xg[le], ew_ref[le, :, 0:d_half].astype(jnp.bfloat16),
                preferred_element_type=jnp.float32,
            )
        commA[0, :, :] = pA.astype(jnp.bfloat16)
        rA1 = exchange(commA, 0, 1, sendA, recvA, px, 0)
        rA1.start()

        pB = jnp.zeros((n_tok, d_half), jnp.float32)
        for le in range(E_LOCAL):
            pB = pB + jnp.dot(
                xg[le], ew_ref[le, :, d_half:d_ff].astype(jnp.bfloat16),
                preferred_element_type=jnp.float32,
            )
        commB[0, :, :] = pB.astype(jnp.bfloat16)
        rB1 = exchange(commB, 0, 1, sendB, recvB, py, 0)
        rB1.start()

        shared = jnp.dot(xv16, sw_ref[:, :].astype(jnp.bfloat16),
                         preferred_element_type=jnp.float32)

        rA1.wait_recv()
        commA[2, :, :] = commA[0, :, :] + commA[1, :, :]
        rA2 = exchange(commA, 2, 3, sendA, recvA, py, 1)
        rA2.start()

        rB1.wait_recv()
        commB[2, :, :] = commB[0, :, :] + commB[1, :, :]
        rB2 = exchange(commB, 2, 3, sendB, recvB, px, 1)
        rB2.start()

        rA2.wait_recv()
        out_ref[:, 0:d_half] = shared[:, 0:d_half] + (
            commA[2, :, :] + commA[3, :, :]
        ).astype(jnp.float32)
        rB2.wait_recv()
        out_ref[:, d_half:d_ff] = shared[:, d_half:d_ff] + (
            commB[2, :, :] + commB[3, :, :]
        ).astype(jnp.float32)

        rA1.wait_send()
        rB1.wait_send()
        rA2.wait_send()
        rB2.wait_send()

    return pl.pallas_call(
        body,
        out_shape=jax.ShapeDtypeStruct((n_tok, d_ff), jnp.float32),
        in_specs=[pl.BlockSpec(memory_space=pltpu.VMEM)] * 5,
        out_specs=pl.BlockSpec(memory_space=pltpu.VMEM),
        scratch_shapes=[
            pltpu.VMEM((4, n_tok, d_half), jnp.bfloat16),
            pltpu.VMEM((4, n_tok, d_half), jnp.bfloat16),
            pltpu.SemaphoreType.DMA((2,)),
            pltpu.SemaphoreType.DMA((2,)),
            pltpu.SemaphoreType.DMA((2,)),
            pltpu.SemaphoreType.DMA((2,)),
        ],
        compiler_params=pltpu.CompilerParams(collective_id=0),
    )(x, router_W, route_idx, expert_W, shared_W)
